# Initial kernel scaffold; baseline (speedup 1.0000x reference)
import functools

import jax
import jax.numpy as jnp
from jax import lax
from jax.experimental import pallas as pl
from jax.experimental.pallas import tpu as pltpu
from jax.experimental.pallas import tpu_sc as plsc

N = 10000
D = 128
E = 320000
NC = 2
NS = 16
NW = NC * NS
C = 128
K = (E + NW * C - 1) // (NW * C)
EPT = K * C
EP = NW * EPT
RPS = N // NS
ND = 10112
DPS = ND // NS

_mesh = plsc.VectorSubcoreMesh(core_axis_name="c", subcore_axis_name="s")


@functools.partial(
    pl.kernel,
    out_type=jax.ShapeDtypeStruct((NC, ND), jnp.float32),
    mesh=_mesh,
    scratch_types=[
        pltpu.VMEM((K, C), jnp.int32),
        pltpu.VMEM((K, C), jnp.float32),
        pltpu.VMEM((640,), jnp.float32),
        pltpu.VMEM_SHARED((ND,), jnp.float32),
    ],
)
def _sc_deg(dst_hbm, w_hbm, out_hbm, dst_v, w_v, zbuf, acc):
    c = lax.axis_index("c")
    s = lax.axis_index("s")
    wid = s * NC + c

    zero16 = jnp.zeros((16,), jnp.float32)

    def zfill(i, _):
        zbuf[pl.ds(i * 16, 16)] = zero16
        return 0

    lax.fori_loop(0, 40, zfill, 0)
    pltpu.sync_copy(zbuf.at[pl.ds(0, DPS)], acc.at[pl.ds(s * DPS, DPS)])
    plsc.subcore_barrier()

    pltpu.sync_copy(dst_hbm.at[wid], dst_v)
    pltpu.sync_copy(w_hbm.at[wid], w_v)

    def chunk(k, _):
        pltpu.sync_copy(w_v.at[k], acc.at[dst_v.at[k]], add=True)
        return 0

    lax.fori_loop(0, K, chunk, 0)
    plsc.subcore_barrier()
    pltpu.sync_copy(acc.at[pl.ds(s * DPS, DPS)], out_hbm.at[c, pl.ds(s * DPS, DPS)])


@functools.partial(
    pl.kernel,
    out_type=jax.ShapeDtypeStruct((NC, N, D), jnp.float32),
    mesh=_mesh,
    scratch_types=[
        pltpu.VMEM((K, C), jnp.int32),
        pltpu.VMEM((K, C), jnp.int32),
        pltpu.VMEM((K, C), jnp.float32),
        pltpu.VMEM((C, D), jnp.float32),
        pltpu.VMEM((125, D), jnp.float32),
        pltpu.VMEM_SHARED((N, D), jnp.float32),
    ],
)
def _sc_agg(g_hbm, src_hbm, dst_hbm, w_hbm, out_hbm, src_v, dst_v, w_v, rows, zbuf, acc):
    c = lax.axis_index("c")
    s = lax.axis_index("s")
    wid = s * NC + c

    zero16 = jnp.zeros((16,), jnp.float32)

    def zfill(i, _):
        for j in range(D // 16):
            zbuf[i, pl.ds(j * 16, 16)] = zero16
        return 0

    lax.fori_loop(0, 125, zfill, 0)
    for r in range(RPS // 125):
        pltpu.sync_copy(zbuf, acc.at[pl.ds(s * RPS + r * 125, 125)])
    plsc.subcore_barrier()

    pltpu.sync_copy(src_hbm.at[wid], src_v)
    pltpu.sync_copy(dst_hbm.at[wid], dst_v)
    pltpu.sync_copy(w_hbm.at[wid], w_v)

    def chunk(k, _):
        pltpu.sync_copy(g_hbm.at[src_v.at[k]], rows)

        def edge(i, _):
            wv = w_v[k, i]
            for j in range(D // 16):
                sl = pl.ds(j * 16, 16)
                rows[i, sl] = rows[i, sl] * wv
            return 0

        lax.fori_loop(0, C, edge, 0)
        pltpu.sync_copy(rows, acc.at[dst_v.at[k]], add=True)
        return 0

    lax.fori_loop(0, K, chunk, 0)
    plsc.subcore_barrier()
    pltpu.sync_copy(acc.at[pl.ds(s * RPS, RPS)], out_hbm.at[c, pl.ds(s * RPS, RPS)])


def _tc_first_body(p0_ref, p1_ref, x_ref, w1_ref, dis_ref, h1_ref, g1_ref):
    deg = p0_ref[pl.ds(0, N), :] + p1_ref[pl.ds(0, N), :] + 1.0
    dis = jnp.where(deg > 0, lax.rsqrt(jnp.where(deg > 0, deg, 1.0)), 0.0)
    dis_ref[...] = dis
    h1 = jnp.dot(x_ref[...], w1_ref[...], preferred_element_type=jnp.float32,
                 precision=lax.Precision.HIGHEST)
    h1_ref[...] = h1
    g1_ref[...] = h1 * dis


def _tc_first(p0, p1, x, W1):
    return pl.pallas_call(
        _tc_first_body,
        out_shape=(
            jax.ShapeDtypeStruct((N, 1), jnp.float32),
            jax.ShapeDtypeStruct((N, D), jnp.float32),
            jax.ShapeDtypeStruct((N, D), jnp.float32),
        ),
    )(p0, p1, x, W1)


def _tc_mid_body(a0_ref, a1_ref, dis_ref, h_ref, b_ref, w2_ref, h2_ref, g2_ref):
    dis = dis_ref[...]
    out1 = dis * (a0_ref[...] + a1_ref[...]) + h_ref[...] * (dis * dis) + b_ref[...]
    h2 = jnp.dot(out1, w2_ref[...], preferred_element_type=jnp.float32,
                 precision=lax.Precision.HIGHEST)
    h2_ref[...] = h2
    g2_ref[...] = h2 * dis


def _tc_mid(a0, a1, dis, h1, b1, W2):
    return pl.pallas_call(
        _tc_mid_body,
        out_shape=(
            jax.ShapeDtypeStruct((N, D), jnp.float32),
            jax.ShapeDtypeStruct((N, D), jnp.float32),
        ),
    )(a0, a1, dis, h1, b1, W2)


def _tc_last_body(a0_ref, a1_ref, dis_ref, h_ref, b_ref, out_ref):
    dis = dis_ref[...]
    out_ref[...] = (dis * (a0_ref[...] + a1_ref[...])
                    + h_ref[...] * (dis * dis) + b_ref[...])


def _tc_last(a0, a1, dis, h2, b2):
    return pl.pallas_call(
        _tc_last_body,
        out_shape=jax.ShapeDtypeStruct((N, D), jnp.float32),
    )(a0, a1, dis, h2, b2)


def kernel(x, edge_index, batch, edge_weight, W1, b1, W2, b2):
    del batch
    pad = EP - E
    src = jnp.concatenate([edge_index[0], jnp.zeros((pad,), jnp.int32)]).reshape(NW, K, C)
    dst = jnp.concatenate([edge_index[1], jnp.zeros((pad,), jnp.int32)]).reshape(NW, K, C)
    w = jnp.concatenate([edge_weight, jnp.zeros((pad,), jnp.float32)]).reshape(NW, K, C)

    deg_parts = _sc_deg(dst, w)
    p0 = deg_parts[0].reshape(ND, 1)
    p1 = deg_parts[1].reshape(ND, 1)

    dis, h1, g1 = _tc_first(p0, p1, x, W1)

    agg1 = _sc_agg(g1, src, dst, w)
    h2, g2 = _tc_mid(agg1[0], agg1[1], dis, h1, b1.reshape(1, D), W2)

    agg2 = _sc_agg(g2, src, dst, w)
    return _tc_last(agg2[0], agg2[1], dis, h2, b2.reshape(1, D))


# speedup vs baseline: 11.6965x; 11.6965x over previous
import functools

import jax
import jax.numpy as jnp
from jax import lax
from jax.experimental import pallas as pl
from jax.experimental.pallas import tpu as pltpu
from jax.experimental.pallas import tpu_sc as plsc

N = 10000
D = 128
E = 320000
NC = 2
NS = 16
NW = NC * NS
C = 128
K = (E + NW * C - 1) // (NW * C)
EPT = K * C
EP = NW * EPT
NP = 10240
RPS = NP // NS
ND = NP
DPS = ND // NS

_mesh = plsc.VectorSubcoreMesh(core_axis_name="c", subcore_axis_name="s")


@functools.partial(
    pl.kernel,
    out_type=jax.ShapeDtypeStruct((NC * ND,), jnp.float32),
    mesh=_mesh,
    scratch_types=[
        pltpu.VMEM((K, C), jnp.int32),
        pltpu.VMEM((K, C), jnp.float32),
        pltpu.VMEM((DPS,), jnp.float32),
        pltpu.VMEM_SHARED((ND,), jnp.float32),
    ],
)
def _sc_deg(dst_hbm, w_hbm, out_hbm, dst_v, w_v, zbuf, acc):
    c = lax.axis_index("c")
    s = lax.axis_index("s")
    wid = s * NC + c

    zero16 = jnp.zeros((16,), jnp.float32)

    def zfill(i, _):
        zbuf[pl.ds(i * 16, 16)] = zero16
        return 0

    lax.fori_loop(0, 40, zfill, 0)
    pltpu.sync_copy(zbuf, acc.at[pl.ds(s * DPS, DPS)])
    plsc.subcore_barrier()

    pltpu.sync_copy(dst_hbm.at[wid], dst_v)
    pltpu.sync_copy(w_hbm.at[wid], w_v)

    def chunk(k, _):
        pltpu.sync_copy(w_v.at[k], acc.at[dst_v.at[k]], add=True)
        return 0

    lax.fori_loop(0, K, chunk, 0)
    plsc.subcore_barrier()
    pltpu.sync_copy(acc.at[pl.ds(s * DPS, DPS)], out_hbm.at[pl.ds(c * ND + s * DPS, DPS)])


@functools.partial(
    pl.kernel,
    out_type=jax.ShapeDtypeStruct((NC * NP, D), jnp.float32),
    mesh=_mesh,
    scratch_types=[
        pltpu.VMEM((K, C), jnp.int32),
        pltpu.VMEM((K, C), jnp.int32),
        pltpu.VMEM((K, C), jnp.float32),
        pltpu.VMEM((C, D), jnp.float32),
        pltpu.VMEM_SHARED((NP, D), jnp.float32),
    ],
)
def _sc_agg(g_hbm, src_hbm, dst_hbm, w_hbm, out_hbm, src_v, dst_v, w_v, rows, acc):
    c = lax.axis_index("c")
    s = lax.axis_index("s")
    wid = s * NC + c

    zero16 = jnp.zeros((16,), jnp.float32)

    def zfill(i, _):
        for j in range(D // 16):
            rows[i, pl.ds(j * 16, 16)] = zero16
        return 0

    lax.fori_loop(0, C, zfill, 0)
    for r in range(RPS // C):
        pltpu.sync_copy(rows, acc.at[pl.ds(s * RPS + r * C, C)])
    plsc.subcore_barrier()

    pltpu.sync_copy(src_hbm.at[wid], src_v)
    pltpu.sync_copy(dst_hbm.at[wid], dst_v)
    pltpu.sync_copy(w_hbm.at[wid], w_v)

    def chunk(k, _):
        pltpu.sync_copy(g_hbm.at[src_v.at[k]], rows)

        def edge_group(gi, _):
            wvec = w_v[k, pl.ds(gi * 16, 16)]
            base = gi * 16
            for l in range(16):
                wv = wvec[l]
                for j in range(D // 16):
                    sl = pl.ds(j * 16, 16)
                    rows[base + l, sl] = rows[base + l, sl] * wv
            return 0

        lax.fori_loop(0, C // 16, edge_group, 0)
        pltpu.sync_copy(rows, acc.at[dst_v.at[k]], add=True)
        return 0

    lax.fori_loop(0, K, chunk, 0)
    plsc.subcore_barrier()
    pltpu.sync_copy(acc.at[pl.ds(s * RPS, RPS)], out_hbm.at[pl.ds(c * NP + s * RPS, RPS)])


def _tc_first_body(p0_ref, p1_ref, x_ref, w1_ref, dis_ref, h1_ref, g1_ref):
    deg = p0_ref[pl.ds(0, N), :] + p1_ref[pl.ds(0, N), :] + 1.0
    dis = jnp.where(deg > 0, lax.rsqrt(jnp.where(deg > 0, deg, 1.0)), 0.0)
    dis_ref[...] = dis
    h1 = jnp.dot(x_ref[...], w1_ref[...], preferred_element_type=jnp.float32,
                 precision=lax.Precision.HIGHEST)
    h1_ref[...] = h1
    g1_ref[...] = h1 * dis


def _tc_first(p0, p1, x, W1):
    return pl.pallas_call(
        _tc_first_body,
        out_shape=(
            jax.ShapeDtypeStruct((N, 1), jnp.float32),
            jax.ShapeDtypeStruct((N, D), jnp.float32),
            jax.ShapeDtypeStruct((N, D), jnp.float32),
        ),
    )(p0, p1, x, W1)


def _tc_mid_body(a0_ref, a1_ref, dis_ref, h_ref, b_ref, w2_ref, h2_ref, g2_ref):
    dis = dis_ref[...]
    out1 = dis * (a0_ref[...] + a1_ref[...]) + h_ref[...] * (dis * dis) + b_ref[...]
    h2 = jnp.dot(out1, w2_ref[...], preferred_element_type=jnp.float32,
                 precision=lax.Precision.HIGHEST)
    h2_ref[...] = h2
    g2_ref[...] = h2 * dis


def _tc_mid(a0, a1, dis, h1, b1, W2):
    return pl.pallas_call(
        _tc_mid_body,
        out_shape=(
            jax.ShapeDtypeStruct((N, D), jnp.float32),
            jax.ShapeDtypeStruct((N, D), jnp.float32),
        ),
    )(a0, a1, dis, h1, b1, W2)


def _tc_last_body(a0_ref, a1_ref, dis_ref, h_ref, b_ref, out_ref):
    dis = dis_ref[...]
    out_ref[...] = (dis * (a0_ref[...] + a1_ref[...])
                    + h_ref[...] * (dis * dis) + b_ref[...])


def _tc_last(a0, a1, dis, h2, b2):
    return pl.pallas_call(
        _tc_last_body,
        out_shape=jax.ShapeDtypeStruct((N, D), jnp.float32),
    )(a0, a1, dis, h2, b2)


def kernel(x, edge_index, batch, edge_weight, W1, b1, W2, b2):
    del batch
    pad = EP - E
    src = jnp.concatenate([edge_index[0], jnp.zeros((pad,), jnp.int32)]).reshape(NW, K, C)
    dst = jnp.concatenate([edge_index[1], jnp.zeros((pad,), jnp.int32)]).reshape(NW, K, C)
    w = jnp.concatenate([edge_weight, jnp.zeros((pad,), jnp.float32)]).reshape(NW, K, C)

    deg_parts = _sc_deg(dst, w)
    p0 = deg_parts[:ND].reshape(ND, 1)
    p1 = deg_parts[ND:].reshape(ND, 1)

    dis, h1, g1 = _tc_first(p0, p1, x, W1)

    agg1 = _sc_agg(g1, src, dst, w)
    h2, g2 = _tc_mid(agg1[:N], agg1[NP:NP + N], dis, h1, b1.reshape(1, D), W2)

    agg2 = _sc_agg(g2, src, dst, w)
    return _tc_last(agg2[:N], agg2[NP:NP + N], dis, h2, b2.reshape(1, D))
